# baseline (speedup 1.0000x reference)
import jax
import jax.numpy as jnp
from jax.experimental import pallas as pl
from jax.experimental.pallas import tpu as pltpu

_ALPHA = 4
_IDX = (0, 4, 8, 12, 16, 21, 25, 29, 33, 37, 42, 46, 50, 54, 58, 63)


def _body(x_ref, slow_ref, fast_ref, sem_fast, sem_slow):
    C = x_ref.shape[0]
    fast_cps = [
        pltpu.make_async_copy(x_ref.at[c], fast_ref.at[c], sem_fast)
        for c in range(C)
    ]
    slow_cps = [
        pltpu.make_async_copy(x_ref.at[c, _IDX[j]], slow_ref.at[c, j], sem_slow)
        for c in range(C)
        for j in range(len(_IDX))
    ]
    for cp in fast_cps:
        cp.start()
    for cp in slow_cps:
        cp.start()
    for cp in fast_cps:
        cp.wait()
    for cp in slow_cps:
        cp.wait()


def kernel(frames):
    C, T, H, W = frames.shape
    Ts = T // _ALPHA
    slow, fast = pl.pallas_call(
        _body,
        in_specs=[pl.BlockSpec(memory_space=pl.ANY)],
        out_specs=[
            pl.BlockSpec(memory_space=pl.ANY),
            pl.BlockSpec(memory_space=pl.ANY),
        ],
        out_shape=[
            jax.ShapeDtypeStruct((C, Ts, H, W), frames.dtype),
            jax.ShapeDtypeStruct((C, T, H, W), frames.dtype),
        ],
        scratch_shapes=[pltpu.SemaphoreType.DMA, pltpu.SemaphoreType.DMA],
    )(frames)
    return (slow, fast)


# speedup vs baseline: 1.0575x; 1.0575x over previous
import jax
import jax.numpy as jnp
from jax.experimental import pallas as pl
from jax.experimental.pallas import tpu as pltpu

_ALPHA = 4
_IDX = (0, 4, 8, 12, 16, 21, 25, 29, 33, 37, 42, 46, 50, 54, 58, 63)
_C, _T, _H, _W = 3, 64, 224, 224
_HW = _H * _W


def _body(x_ref, slow_ref, fast_ref, sem_fast, sem_slow):
    fast_cp = pltpu.make_async_copy(x_ref, fast_ref, sem_fast)
    slow_cps = [
        pltpu.make_async_copy(
            x_ref.at[pl.ds((c * _T + _IDX[j]) * _HW, _HW)],
            slow_ref.at[pl.ds((c * len(_IDX) + j) * _HW, _HW)],
            sem_slow,
        )
        for c in range(_C)
        for j in range(len(_IDX))
    ]
    fast_cp.start()
    for cp in slow_cps:
        cp.start()
    fast_cp.wait()
    for cp in slow_cps:
        cp.wait()


def kernel(frames):
    C, T, H, W = frames.shape
    Ts = T // _ALPHA
    x = frames.reshape(C * T * H * W)
    slow, fast = pl.pallas_call(
        _body,
        in_specs=[pl.BlockSpec(memory_space=pl.ANY)],
        out_specs=[
            pl.BlockSpec(memory_space=pl.ANY),
            pl.BlockSpec(memory_space=pl.ANY),
        ],
        out_shape=[
            jax.ShapeDtypeStruct((C * Ts * H * W,), frames.dtype),
            jax.ShapeDtypeStruct((C * T * H * W,), frames.dtype),
        ],
        scratch_shapes=[pltpu.SemaphoreType.DMA, pltpu.SemaphoreType.DMA],
    )(x)
    return (slow.reshape(C, Ts, H, W), fast.reshape(C, T, H, W))


# speedup vs baseline: 44.7730x; 42.3367x over previous
import jax
import jax.numpy as jnp
from jax.experimental import pallas as pl
from jax.experimental.pallas import tpu as pltpu

_ALPHA = 4
_IDX = (0, 4, 8, 12, 16, 21, 25, 29, 33, 37, 42, 46, 50, 54, 58, 63)
_TBLK = 16
_SEL = 4


def _body(x_ref, slow_ref, fast_ref, sem_fast, sem_slow):
    c = pl.program_id(0)
    w = pl.program_id(1)
    fast_cp = pltpu.make_async_copy(
        x_ref.at[0], fast_ref.at[c, pl.ds(w * _TBLK, _TBLK)], sem_fast
    )
    slow_cps = []
    for k in range(_SEL):
        j = w * _SEL + k
        g = (63 * j) // 15 - w * _TBLK
        slow_cps.append(
            pltpu.make_async_copy(x_ref.at[0, g], slow_ref.at[c, j], sem_slow)
        )
    fast_cp.start()
    for cp in slow_cps:
        cp.start()
    fast_cp.wait()
    for cp in slow_cps:
        cp.wait()


def kernel(frames):
    C, T, H, W = frames.shape
    Ts = T // _ALPHA
    slow, fast = pl.pallas_call(
        _body,
        grid=(C, T // _TBLK),
        in_specs=[
            pl.BlockSpec((1, _TBLK, H, W), lambda c, w: (c, w, 0, 0)),
        ],
        out_specs=[
            pl.BlockSpec(memory_space=pl.ANY),
            pl.BlockSpec(memory_space=pl.ANY),
        ],
        out_shape=[
            jax.ShapeDtypeStruct((C, Ts, H, W), frames.dtype),
            jax.ShapeDtypeStruct((C, T, H, W), frames.dtype),
        ],
        scratch_shapes=[pltpu.SemaphoreType.DMA, pltpu.SemaphoreType.DMA],
    )(frames)
    return (slow, fast)


# speedup vs baseline: 50.8326x; 1.1353x over previous
import jax
import jax.numpy as jnp
from jax.experimental import pallas as pl
from jax.experimental.pallas import tpu as pltpu

_ALPHA = 4
_IDX = (0, 4, 8, 12, 16, 21, 25, 29, 33, 37, 42, 46, 50, 54, 58, 63)
_TBLK = 32
_SEL = 8


def _body(x_ref, slow_ref, fast_ref, sem_fast, sem_slow):
    c = pl.program_id(0)
    w = pl.program_id(1)
    fast_cp = pltpu.make_async_copy(
        x_ref.at[0], fast_ref.at[c, pl.ds(w * _TBLK, _TBLK)], sem_fast
    )
    slow_cps = []
    for k in range(_SEL):
        j = w * _SEL + k
        g = (63 * j) // 15 - w * _TBLK
        slow_cps.append(
            pltpu.make_async_copy(x_ref.at[0, g], slow_ref.at[c, j], sem_slow)
        )
    fast_cp.start()
    for cp in slow_cps:
        cp.start()
    fast_cp.wait()
    for cp in slow_cps:
        cp.wait()


def kernel(frames):
    C, T, H, W = frames.shape
    Ts = T // _ALPHA
    slow, fast = pl.pallas_call(
        _body,
        grid=(C, T // _TBLK),
        in_specs=[
            pl.BlockSpec((1, _TBLK, H, W), lambda c, w: (c, w, 0, 0)),
        ],
        out_specs=[
            pl.BlockSpec(memory_space=pl.ANY),
            pl.BlockSpec(memory_space=pl.ANY),
        ],
        out_shape=[
            jax.ShapeDtypeStruct((C, Ts, H, W), frames.dtype),
            jax.ShapeDtypeStruct((C, T, H, W), frames.dtype),
        ],
        scratch_shapes=[pltpu.SemaphoreType.DMA, pltpu.SemaphoreType.DMA],
    )(frames)
    return (slow, fast)


# speedup vs baseline: 52.4128x; 1.0311x over previous
import jax
import jax.numpy as jnp
from jax.experimental import pallas as pl
from jax.experimental.pallas import tpu as pltpu

_ALPHA = 4
_IDX = (0, 4, 8, 12, 16, 21, 25, 29, 33, 37, 42, 46, 50, 54, 58, 63)
_TBLK = 64
_SEL = 16


def _body(x_ref, slow_ref, fast_ref, sem_fast, sem_slow):
    c = pl.program_id(0)
    w = pl.program_id(1)
    fast_cp = pltpu.make_async_copy(
        x_ref.at[0], fast_ref.at[c, pl.ds(w * _TBLK, _TBLK)], sem_fast
    )
    slow_cps = []
    for k in range(_SEL):
        j = w * _SEL + k
        g = (63 * j) // 15 - w * _TBLK
        slow_cps.append(
            pltpu.make_async_copy(x_ref.at[0, g], slow_ref.at[c, j], sem_slow)
        )
    fast_cp.start()
    for cp in slow_cps:
        cp.start()
    fast_cp.wait()
    for cp in slow_cps:
        cp.wait()


def kernel(frames):
    C, T, H, W = frames.shape
    Ts = T // _ALPHA
    slow, fast = pl.pallas_call(
        _body,
        grid=(C, T // _TBLK),
        in_specs=[
            pl.BlockSpec((1, _TBLK, H, W), lambda c, w: (c, w, 0, 0)),
        ],
        out_specs=[
            pl.BlockSpec(memory_space=pl.ANY),
            pl.BlockSpec(memory_space=pl.ANY),
        ],
        out_shape=[
            jax.ShapeDtypeStruct((C, Ts, H, W), frames.dtype),
            jax.ShapeDtypeStruct((C, T, H, W), frames.dtype),
        ],
        scratch_shapes=[pltpu.SemaphoreType.DMA, pltpu.SemaphoreType.DMA],
    )(frames)
    return (slow, fast)
